# Initial kernel scaffold; baseline (speedup 1.0000x reference)
import jax
import jax.numpy as jnp
from jax.experimental import pallas as pl


def kernel(x, W, b, gamma, beta):
    raise NotImplementedError("write your pallas kernel here")


# speedup vs baseline: 5.7953x; 5.7953x over previous
import functools

import jax
import jax.numpy as jnp
from jax import lax
from jax.experimental import pallas as pl
from jax.experimental.pallas import tpu as pltpu

K_NB = 16
_BIG = 1e9


def _prep_kernel(x_ref, xn_ref, md_ref):
    x = x_ref[0]
    C, N = x.shape
    nrm = jnp.sqrt(jnp.sum(x * x, axis=0, keepdims=True))
    xn = x * (1.0 / jnp.maximum(nrm, 1e-12))
    sq = jnp.sum(xn * xn, axis=0, keepdims=True)
    g = lax.dot_general(xn, xn, (((0,), (0,)), ((), ())),
                        preferred_element_type=jnp.float32)
    d2 = jnp.transpose(sq) + sq - 2.0 * g
    d2 = jnp.maximum(d2, 0.0)
    rowid = lax.broadcasted_iota(jnp.int32, (N, N), 0)
    colid = lax.broadcasted_iota(jnp.int32, (N, N), 1)
    d2 = jnp.where(rowid == colid, _BIG, d2)
    xnb = xn.astype(jnp.bfloat16)

    def body(_, carry):
        d2c, md = carry
        m = jnp.min(d2c, axis=1, keepdims=True)
        cand = jnp.where(d2c == m, colid, N)
        first = jnp.min(cand, axis=1, keepdims=True)
        oh = colid == first
        nb = lax.dot_general(xnb, oh.astype(jnp.bfloat16),
                             (((1,), (1,)), ((), ())),
                             preferred_element_type=jnp.float32)
        md = jnp.maximum(md, jnp.abs(xn - nb))
        d2c = jnp.where(oh, _BIG, d2c)
        return d2c, md

    _, md = lax.fori_loop(0, K_NB, body, (d2, jnp.zeros_like(xn)))
    xn_ref[0] = xn
    md_ref[0] = md


def _conv_kernel(we_ref, wo_ref, bias_ref, xn_ref, md_ref,
                 y_ref, s1_ref, s2_ref):
    b = pl.program_id(0)
    y = lax.dot_general(we_ref[...], xn_ref[0], (((1,), (0,)), ((), ())),
                        preferred_element_type=jnp.float32)
    y = y + lax.dot_general(wo_ref[...], md_ref[0], (((1,), (0,)), ((), ())),
                            preferred_element_type=jnp.float32)
    y = y + bias_ref[...]
    y_ref[0] = y
    ps1 = jnp.sum(y, axis=1, keepdims=True)
    ps2 = jnp.sum(y * y, axis=1, keepdims=True)

    @pl.when(b == 0)
    def _():
        s1_ref[...] = ps1
        s2_ref[...] = ps2

    @pl.when(b != 0)
    def _():
        s1_ref[...] += ps1
        s2_ref[...] += ps2


def _bn_gelu_kernel(y_ref, s1_ref, s2_ref, gamma_ref, beta_ref, o_ref,
                    *, count):
    mean = s1_ref[...] * (1.0 / count)
    var = s2_ref[...] * (1.0 / count) - mean * mean
    scale = gamma_ref[...] * lax.rsqrt(var + 1e-5)
    shift = beta_ref[...] - mean * scale
    yn = y_ref[0] * scale + shift
    o_ref[0] = yn * 0.5 * (1.0 + lax.erf(yn * 0.7071067811865476))


def kernel(x, W, b, gamma, beta):
    B, C, N = x.shape
    O = W.shape[0]
    we = W[:, 0::2]
    wo = W[:, 1::2]

    xn, md = pl.pallas_call(
        _prep_kernel,
        grid=(B,),
        in_specs=[pl.BlockSpec((1, C, N), lambda i: (i, 0, 0))],
        out_specs=[pl.BlockSpec((1, C, N), lambda i: (i, 0, 0)),
                   pl.BlockSpec((1, C, N), lambda i: (i, 0, 0))],
        out_shape=[jax.ShapeDtypeStruct((B, C, N), jnp.float32),
                   jax.ShapeDtypeStruct((B, C, N), jnp.float32)],
    )(x)

    y, s1, s2 = pl.pallas_call(
        _conv_kernel,
        grid=(B,),
        in_specs=[pl.BlockSpec((O, C), lambda i: (0, 0)),
                  pl.BlockSpec((O, C), lambda i: (0, 0)),
                  pl.BlockSpec((O, 1), lambda i: (0, 0)),
                  pl.BlockSpec((1, C, N), lambda i: (i, 0, 0)),
                  pl.BlockSpec((1, C, N), lambda i: (i, 0, 0))],
        out_specs=[pl.BlockSpec((1, O, N), lambda i: (i, 0, 0)),
                   pl.BlockSpec((O, 1), lambda i: (0, 0)),
                   pl.BlockSpec((O, 1), lambda i: (0, 0))],
        out_shape=[jax.ShapeDtypeStruct((B, O, N), jnp.float32),
                   jax.ShapeDtypeStruct((O, 1), jnp.float32),
                   jax.ShapeDtypeStruct((O, 1), jnp.float32)],
    )(we, wo, b.reshape(O, 1), xn, md)

    out = pl.pallas_call(
        functools.partial(_bn_gelu_kernel, count=float(B * N)),
        grid=(B,),
        in_specs=[pl.BlockSpec((1, O, N), lambda i: (i, 0, 0)),
                  pl.BlockSpec((O, 1), lambda i: (0, 0)),
                  pl.BlockSpec((O, 1), lambda i: (0, 0)),
                  pl.BlockSpec((O, 1), lambda i: (0, 0)),
                  pl.BlockSpec((O, 1), lambda i: (0, 0))],
        out_specs=pl.BlockSpec((1, O, N), lambda i: (i, 0, 0)),
        out_shape=jax.ShapeDtypeStruct((B, O, N), jnp.float32),
    )(y, s1, s2, gamma.reshape(O, 1), beta.reshape(O, 1))

    return out.reshape(B, O, N, 1)
